# Initial kernel scaffold; baseline (speedup 1.0000x reference)
import functools

import jax
import jax.numpy as jnp
from jax import lax
from jax.experimental import pallas as pl
from jax.experimental.pallas import tpu as pltpu
from jax.experimental.pallas import tpu_sc as plsc

_R = 1000
_RPAD = 1024
_NR = 8
_ER = 8
_D = 32
_OUT = 32
_NL = 3
_NES = 64
_NNS = 32
_NEXT = 16

_NC = 2
_NS = 16
_NW = _NC * _NS
_CHUNK = 80


def _table_body(esT, nsT, eiT, evT, en0T, en1T, atT, bdT, exT,
                WlT, blT, WoT, boT, outT):
    f32 = jnp.float32

    def onehot(idx_row, k):
        ks = lax.broadcasted_iota(jnp.int32, (k, _RPAD), 0)
        return (idx_row == ks).astype(f32)

    edge_h = []
    for e in range(_ER):
        edge_h.append(jnp.dot(atT[...], onehot(esT[e:e + 1, :], _NES),
                              preferred_element_type=f32))
    node_h = []
    for n in range(_NR):
        hb = jnp.dot(bdT[...], onehot(nsT[n:n + 1, :], _NNS),
                     preferred_element_type=f32)
        hx = jnp.dot(exT[...], onehot(eiT[n:n + 1, :], _NEXT),
                     preferred_element_type=f32)
        node_h.append(hb + evT[n:n + 1, :] * hx)

    A = []
    for e in range(_ER):
        e0 = en0T[e:e + 1, :]
        e1 = en1T[e:e + 1, :]
        A.append([(e0 == n).astype(f32) + (e1 == n).astype(f32)
                  for n in range(_NR)])

    acc = jnp.zeros((_OUT, _RPAD), f32)
    for l in range(_NL):
        Wl = WlT[_D * l:_D * (l + 1), :]
        Wo = WoT[_D * l:_D * (l + 1), :]
        bl = blT[:, l:l + 1]
        bo = boT[:, l:l + 1]
        v_e = []
        for e in range(_ER):
            m = edge_h[e]
            for n in range(_NR):
                m = m + A[e][n] * node_h[n]
            v_e.append(m)
        v_n = []
        for n in range(_NR):
            m = node_h[n]
            for e in range(_ER):
                m = m + A[e][n] * edge_h[e]
            v_n.append(m)
        for v in v_e + v_n:
            acc = acc + jnp.maximum(
                jnp.dot(Wo, v, preferred_element_type=f32) + bo, 0.0)
        for e in range(_ER):
            edge_h[e] = jnp.maximum(
                jnp.dot(Wl, v_e[e], preferred_element_type=f32) + bl, 0.0)
        for n in range(_NR):
            node_h[n] = jnp.maximum(
                jnp.dot(Wl, v_n[n], preferred_element_type=f32) + bl, 0.0)

    lane = lax.broadcasted_iota(jnp.int32, (_OUT, _RPAD), 1)
    outT[...] = jnp.where(lane < _R, acc, 0.0)


def _compute_tableT(esT, nsT, eiT, evT, en0T, en1T, atT, bdT, exT,
                    WlT, blT, WoT, boT):
    return pl.pallas_call(
        _table_body,
        out_shape=jax.ShapeDtypeStruct((_OUT, _RPAD), jnp.float32),
    )(esT, nsT, eiT, evT, en0T, en1T, atT, bdT, exT, WlT, blT, WoT, boT)


def _sc_gather(table, idx_flat, tok):
    bpw = tok // _NW
    mesh = plsc.VectorSubcoreMesh(core_axis_name="c", subcore_axis_name="s")

    @functools.partial(
        pl.kernel,
        out_type=jax.ShapeDtypeStruct((tok, _OUT), jnp.float32),
        mesh=mesh,
        scratch_types=[
            pltpu.VMEM((bpw,), jnp.int32),
            pltpu.VMEM((bpw, _OUT), jnp.float32),
            pltpu.SemaphoreType.DMA,
        ],
    )
    def gather_k(table_hbm, idx_hbm, out_hbm, idx_v, rows_v, sem):
        wid = lax.axis_index("s") * _NC + lax.axis_index("c")
        base = wid * bpw
        pltpu.sync_copy(idx_hbm.at[pl.ds(base, bpw)], idx_v)
        copies = []
        for c in range(bpw // _CHUNK):
            copies.append(pltpu.async_copy(
                table_hbm.at[idx_v.at[pl.ds(c * _CHUNK, _CHUNK)]],
                rows_v.at[pl.ds(c * _CHUNK, _CHUNK)],
                sem))
        for cp in copies:
            cp.wait()
        pltpu.sync_copy(rows_v, out_hbm.at[pl.ds(base, bpw)])

    return gather_k(table, idx_flat)


def kernel(prod_rule_idx_seq, atom_embed, bond_embed, ext_id_embed,
           W_l2l, b_l2l, W_l2o, b_l2o,
           rule_edge_sym, rule_node_sym, rule_ext_id, rule_ext_valid,
           rule_edge_nodes):
    b, l = prod_rule_idx_seq.shape
    tok = b * l

    def padT(x):
        return jnp.pad(x, ((0, _RPAD - _R), (0, 0))).T

    esT = padT(rule_edge_sym).astype(jnp.int32)
    nsT = padT(rule_node_sym).astype(jnp.int32)
    eiT = padT(rule_ext_id).astype(jnp.int32)
    evT = padT(rule_ext_valid).astype(jnp.float32)
    en0T = padT(rule_edge_nodes[:, :, 0]).astype(jnp.int32)
    en1T = padT(rule_edge_nodes[:, :, 1]).astype(jnp.int32)

    atT = atom_embed.T
    bdT = bond_embed.T
    exT = ext_id_embed.T
    WlT = jnp.concatenate([W_l2l[i].T for i in range(_NL)], axis=0)
    WoT = jnp.concatenate([W_l2o[i].T for i in range(_NL)], axis=0)
    blT = b_l2l.T
    boT = b_l2o.T

    tableT = _compute_tableT(esT, nsT, eiT, evT, en0T, en1T, atT, bdT, exT,
                             WlT, blT, WoT, boT)
    table = tableT.T

    idx_flat = prod_rule_idx_seq.reshape(tok).astype(jnp.int32)
    out_flat = _sc_gather(table, idx_flat, tok)
    return out_flat.reshape(b, l, _OUT)


# speedup vs baseline: 630.8983x; 630.8983x over previous
import functools

import jax
import jax.numpy as jnp
from jax import lax
from jax.experimental import pallas as pl
from jax.experimental.pallas import tpu as pltpu
from jax.experimental.pallas import tpu_sc as plsc

_R = 1000
_RPAD = 1024
_NR = 8
_ER = 8
_D = 32
_OUT = 32
_NL = 3
_NES = 64
_NNS = 32
_NEXT = 16

_NC = 2
_NS = 16
_NW = _NC * _NS
_CHUNK = 80


def _table_body(esT, nsT, eiT, evT, en0T, en1T, atT, bdT, exT,
                WlT, blT, WoT, boT, outT):
    f32 = jnp.float32

    def onehot(idx_row, k):
        ks = lax.broadcasted_iota(jnp.int32, (k, _RPAD), 0)
        return (idx_row == ks).astype(f32)

    edge_h = []
    for e in range(_ER):
        edge_h.append(jnp.dot(atT[...], onehot(esT[e:e + 1, :], _NES),
                              preferred_element_type=f32))
    node_h = []
    for n in range(_NR):
        hb = jnp.dot(bdT[...], onehot(nsT[n:n + 1, :], _NNS),
                     preferred_element_type=f32)
        hx = jnp.dot(exT[...], onehot(eiT[n:n + 1, :], _NEXT),
                     preferred_element_type=f32)
        node_h.append(hb + evT[n:n + 1, :] * hx)

    A = []
    for e in range(_ER):
        e0 = en0T[e:e + 1, :]
        e1 = en1T[e:e + 1, :]
        A.append([(e0 == n).astype(f32) + (e1 == n).astype(f32)
                  for n in range(_NR)])

    acc = jnp.zeros((_OUT, _RPAD), f32)
    for l in range(_NL):
        Wl = WlT[_D * l:_D * (l + 1), :]
        Wo = WoT[_D * l:_D * (l + 1), :]
        bl = blT[:, l:l + 1]
        bo = boT[:, l:l + 1]
        v_e = []
        for e in range(_ER):
            m = edge_h[e]
            for n in range(_NR):
                m = m + A[e][n] * node_h[n]
            v_e.append(m)
        v_n = []
        for n in range(_NR):
            m = node_h[n]
            for e in range(_ER):
                m = m + A[e][n] * edge_h[e]
            v_n.append(m)
        for v in v_e + v_n:
            acc = acc + jnp.maximum(
                jnp.dot(Wo, v, preferred_element_type=f32) + bo, 0.0)
        for e in range(_ER):
            edge_h[e] = jnp.maximum(
                jnp.dot(Wl, v_e[e], preferred_element_type=f32) + bl, 0.0)
        for n in range(_NR):
            node_h[n] = jnp.maximum(
                jnp.dot(Wl, v_n[n], preferred_element_type=f32) + bl, 0.0)

    lane = lax.broadcasted_iota(jnp.int32, (_OUT, _RPAD), 1)
    outT[...] = jnp.where(lane < _R, acc, 0.0)


def _compute_tableT(esT, nsT, eiT, evT, en0T, en1T, atT, bdT, exT,
                    WlT, blT, WoT, boT):
    return pl.pallas_call(
        _table_body,
        out_shape=jax.ShapeDtypeStruct((_OUT, _RPAD), jnp.float32),
    )(esT, nsT, eiT, evT, en0T, en1T, atT, bdT, exT, WlT, blT, WoT, boT)


def _sc_gather(table, idx_flat, tok):
    bpw = tok // _NW
    mesh = plsc.VectorSubcoreMesh(core_axis_name="c", subcore_axis_name="s")

    @functools.partial(
        pl.kernel,
        out_type=jax.ShapeDtypeStruct((tok, _OUT), jnp.float32),
        mesh=mesh,
        compiler_params=pltpu.CompilerParams(use_tc_tiling_on_sc=False),
        scratch_types=[
            pltpu.VMEM((bpw,), jnp.int32),
            pltpu.VMEM((bpw, _OUT), jnp.float32),
            pltpu.SemaphoreType.DMA,
        ],
    )
    def gather_k(table_hbm, idx_hbm, out_hbm, idx_v, rows_v, sem):
        wid = lax.axis_index("s") * _NC + lax.axis_index("c")
        base = wid * bpw
        pltpu.sync_copy(idx_hbm.at[pl.ds(base, bpw)], idx_v)
        copies = []
        for c in range(bpw // _CHUNK):
            copies.append(pltpu.async_copy(
                table_hbm.at[idx_v.at[pl.ds(c * _CHUNK, _CHUNK)]],
                rows_v.at[pl.ds(c * _CHUNK, _CHUNK)],
                sem))
        for cp in copies:
            cp.wait()
        pltpu.sync_copy(rows_v, out_hbm.at[pl.ds(base, bpw)])

    return gather_k(table, idx_flat)


def kernel(prod_rule_idx_seq, atom_embed, bond_embed, ext_id_embed,
           W_l2l, b_l2l, W_l2o, b_l2o,
           rule_edge_sym, rule_node_sym, rule_ext_id, rule_ext_valid,
           rule_edge_nodes):
    b, l = prod_rule_idx_seq.shape
    tok = b * l

    def padT(x):
        return jnp.pad(x, ((0, _RPAD - _R), (0, 0))).T

    esT = padT(rule_edge_sym).astype(jnp.int32)
    nsT = padT(rule_node_sym).astype(jnp.int32)
    eiT = padT(rule_ext_id).astype(jnp.int32)
    evT = padT(rule_ext_valid).astype(jnp.float32)
    en0T = padT(rule_edge_nodes[:, :, 0]).astype(jnp.int32)
    en1T = padT(rule_edge_nodes[:, :, 1]).astype(jnp.int32)

    atT = atom_embed.T
    bdT = bond_embed.T
    exT = ext_id_embed.T
    WlT = jnp.concatenate([W_l2l[i].T for i in range(_NL)], axis=0)
    WoT = jnp.concatenate([W_l2o[i].T for i in range(_NL)], axis=0)
    blT = b_l2l.T
    boT = b_l2o.T

    tableT = _compute_tableT(esT, nsT, eiT, evT, en0T, en1T, atT, bdT, exT,
                             WlT, blT, WoT, boT)
    table = tableT.T

    idx_flat = prod_rule_idx_seq.reshape(tok).astype(jnp.int32)
    out_flat = _sc_gather(table, idx_flat, tok)
    return out_flat.reshape(b, l, _OUT)
